# Initial kernel scaffold; baseline (speedup 1.0000x reference)
import jax
import jax.numpy as jnp
from jax.experimental import pallas as pl


def kernel(batched_coordinates):
    raise NotImplementedError("write your pallas kernel here")


# speedup vs baseline: 8.2581x; 8.2581x over previous
import functools

import jax
import jax.numpy as jnp
from jax.experimental import pallas as pl
from jax.experimental.pallas import tpu as pltpu

_K = 32
_EPS = 1e-12
_R = 128
_N = 2048
_SWEEPS = 10


def _b(x):
    return x.astype(jnp.bfloat16).astype(jnp.float32)


def _rot(A, V, p, q):
    apq = A[(p, q)]
    app = A[(p, p)]
    aqq = A[(q, q)]
    denom = 2.0 * apq
    zero = denom == 0.0
    tau = (aqq - app) / jnp.where(zero, 1.0, denom)
    sg = jnp.where(tau >= 0.0, 1.0, -1.0)
    t = sg / (jnp.abs(tau) + jnp.sqrt(1.0 + tau * tau))
    c = 1.0 / jnp.sqrt(1.0 + t * t)
    s = t * c
    c = jnp.where(zero, 1.0, c)
    s = jnp.where(zero, 0.0, s)
    B = dict(A)
    for i in range(3):
        B[(i, p)] = c * A[(i, p)] - s * A[(i, q)]
        B[(i, q)] = s * A[(i, p)] + c * A[(i, q)]
    C = dict(B)
    for j in range(3):
        C[(p, j)] = c * B[(p, j)] - s * B[(q, j)]
        C[(q, j)] = s * B[(p, j)] + c * B[(q, j)]
    W = dict(V)
    for i in range(3):
        W[(i, p)] = c * V[(i, p)] - s * V[(i, q)]
        W[(i, q)] = s * V[(i, p)] + c * V[(i, q)]
    return C, W


def _tile_kernel(c_tile_ref, c_all_ref, ct_ref, out_ref, xs, ys, zs):
    f32 = jnp.float32
    cx_t = c_tile_ref[0, :, 0:1]
    cy_t = c_tile_ref[0, :, 1:2]
    cz_t = c_tile_ref[0, :, 2:3]
    cA = c_all_ref[0]
    cx_r = ct_ref[0, 0:1, :]
    cy_r = ct_ref[0, 1:2, :]
    cz_r = ct_ref[0, 2:3, :]

    sq_t = cx_t * cx_t + cy_t * cy_t + cz_t * cz_t
    sq_r = cx_r * cx_r + cy_r * cy_r + cz_r * cz_r
    dot = (_b(cx_t) * _b(cx_r) + _b(cy_t) * _b(cy_r)
           + _b(cz_t) * _b(cz_r))
    d2 = (sq_t - 2.0 * dot) + sq_r
    d2 = jnp.maximum(d2, 0.0)

    iota = jax.lax.broadcasted_iota(jnp.int32, (_R, _N), 1)
    inf = jnp.float32(jnp.inf)
    for k in range(_K):
        m = jnp.min(d2, axis=1, keepdims=True)
        cand = d2 == m
        idx = jnp.min(jnp.where(cand, iota, _N), axis=1, keepdims=True)
        onehot = iota == idx
        d2 = jnp.where(onehot, inf, d2)
        nb = jax.lax.dot_general(onehot.astype(f32), cA,
                                 (((1,), (0,)), ((), ())),
                                 precision=jax.lax.Precision.HIGHEST,
                                 preferred_element_type=f32)
        xs[:, k:k + 1] = nb[:, 0:1] - cx_t
        ys[:, k:k + 1] = nb[:, 1:2] - cy_t
        zs[:, k:k + 1] = nb[:, 2:3] - cz_t
    m32 = jnp.min(d2, axis=1, keepdims=True)

    X = xs[:, :]
    Y = ys[:, :]
    Z = zs[:, :]
    radius = jnp.sqrt(m32 + _EPS)
    dist = jnp.sqrt(X * X + Y * Y + Z * Z + _EPS)
    w = radius - dist
    wsum = jnp.sum(w, axis=1, keepdims=True) + _EPS

    wX, wY, wZ = _b(w * X), _b(w * Y), _b(w * Z)
    bX, bY, bZ = _b(X), _b(Y), _b(Z)

    def _cc(a, bb):
        return jnp.sum(a * bb, axis=1, keepdims=True) / wsum

    A = {}
    A[(0, 0)] = _cc(wX, bX)
    A[(1, 1)] = _cc(wY, bY)
    A[(2, 2)] = _cc(wZ, bZ)
    A[(0, 1)] = 0.5 * (_cc(wX, bY) + _cc(wY, bX))
    A[(0, 2)] = 0.5 * (_cc(wX, bZ) + _cc(wZ, bX))
    A[(1, 2)] = 0.5 * (_cc(wY, bZ) + _cc(wZ, bY))
    A[(1, 0)] = A[(0, 1)]
    A[(2, 0)] = A[(0, 2)]
    A[(2, 1)] = A[(1, 2)]

    one = jnp.ones_like(wsum)
    nil = jnp.zeros_like(wsum)
    V = {(i, j): (one if i == j else nil) for i in range(3) for j in range(3)}
    for _ in range(_SWEEPS):
        A, V = _rot(A, V, 0, 2)
        A, V = _rot(A, V, 1, 2)
        A, V = _rot(A, V, 0, 1)

    d0, d1, d2e = A[(0, 0)], A[(1, 1)], A[(2, 2)]
    min0 = (d0 <= d1) & (d0 <= d2e)
    min1 = d1 <= d2e
    max2 = (d2e >= d0) & (d2e >= d1)
    max1 = d1 >= d0

    def pick(cond_a, col_a, cond_b, col_b, col_c):
        return [jnp.where(cond_a, V[(i, col_a)],
                          jnp.where(cond_b, V[(i, col_b)], V[(i, col_c)]))
                for i in range(3)]

    z_ax = pick(min0, 0, min1, 1, 2)
    x_ax = pick(max2, 2, max1, 1, 0)

    def disamb(ax):
        dots = (bX * _b(ax[0]) + bY * _b(ax[1]) + bZ * _b(ax[2]))
        pos = jnp.sum(jnp.where(dots >= 0.0, 1.0, 0.0), axis=1, keepdims=True)
        sign = jnp.where(pos >= (_K - pos), 1.0, -1.0)
        return [a * sign for a in ax]

    z_ax = disamb(z_ax)
    x_ax = disamb(x_ax)
    y_ax = [z_ax[1] * x_ax[2] - z_ax[2] * x_ax[1],
            z_ax[2] * x_ax[0] - z_ax[0] * x_ax[2],
            z_ax[0] * x_ax[1] - z_ax[1] * x_ax[0]]

    lx = bX * _b(x_ax[0]) + bY * _b(x_ax[1]) + bZ * _b(x_ax[2])
    ly = bX * _b(y_ax[0]) + bY * _b(y_ax[1]) + bZ * _b(y_ax[2])
    n2 = jnp.sqrt(lx * lx + ly * ly + _EPS)
    scale = dist / n2
    out_ref[0, 0] = lx * scale
    out_ref[0, 1] = ly * scale


@jax.jit
def kernel(batched_coordinates):
    B, N, _ = batched_coordinates.shape
    c = batched_coordinates
    ct = jnp.transpose(c, (0, 2, 1))
    grid = (B, N // _R)
    out = pl.pallas_call(
        _tile_kernel,
        grid=grid,
        in_specs=[
            pl.BlockSpec((1, _R, 3), lambda b, t: (b, t, 0)),
            pl.BlockSpec((1, N, 3), lambda b, t: (b, 0, 0)),
            pl.BlockSpec((1, 3, N), lambda b, t: (b, 0, 0)),
        ],
        out_specs=pl.BlockSpec((1, 2, _R, _K), lambda b, t: (b, 0, t, 0)),
        out_shape=jax.ShapeDtypeStruct((B, 2, N, _K), jnp.float32),
        scratch_shapes=[
            pltpu.VMEM((_R, _K), jnp.float32),
            pltpu.VMEM((_R, _K), jnp.float32),
            pltpu.VMEM((_R, _K), jnp.float32),
        ],
    )(c, c, ct)
    return jnp.transpose(out, (0, 2, 3, 1))
